# baseline (speedup 1.0000x reference)
import functools

import jax
import jax.numpy as jnp
from jax import lax
from jax.experimental import pallas as pl
from jax.experimental.pallas import tpu as pltpu
from jax.experimental.pallas import tpu_sc as plsc

NC, NS, L = 2, 16, 16
NW = NC * NS


def kernel(x, Imagetype, neighbor_list, n):
    B, N, M, K = x.shape
    NT, D = n.shape
    DD = D * D
    TC = N // 128
    JW = M // 4
    PQA = 13
    PQB = DD - PQA
    LANE = K * 128
    ROWJ = DD * TC * LANE

    xt = x.reshape(B, TC, 128, M, K).transpose(0, 3, 1, 4, 2).reshape(-1)
    nlt = neighbor_list.reshape(B, TC, 128, M // 8, 8).transpose(0, 3, 1, 4, 2).reshape(-1)
    itt = Imagetype.reshape(B, TC, 128).transpose(1, 0, 2).reshape(-1)
    n_pad = jnp.pad(n.reshape(NT * D), (0, 256 - NT * D))

    mesh = plsc.VectorSubcoreMesh(core_axis_name="c", subcore_axis_name="s")

    @functools.partial(
        pl.kernel,
        out_type=jax.ShapeDtypeStruct((B * M * DD * TC * K * 128,), jnp.float32),
        mesh=mesh,
        compiler_params=pltpu.CompilerParams(needs_layout_passes=False),
        scratch_types=[
            pltpu.VMEM((JW * TC * K * 128,), jnp.float32),
            pltpu.VMEM((2 * TC * 8 * 128,), jnp.int32),
            pltpu.VMEM((N,), jnp.int32),
            pltpu.VMEM((256,), jnp.float32),
            pltpu.VMEM((JW * TC * 128,), jnp.int32),
            pltpu.VMEM((D * TC * 128,), jnp.float32),
            pltpu.VMEM((PQA * TC * K * 128,), jnp.float32),
            pltpu.VMEM((PQB * TC * K * 128,), jnp.float32),
            pltpu.SemaphoreType.DMA,
            pltpu.SemaphoreType.DMA,
        ],
    )
    def sc_kernel(xt_hbm, nlt_hbm, it_hbm, n_hbm, out_hbm,
                  xst, nlst, itv, nv, tjb, iitab, bufa, bufb, sema, semb):
        wid = lax.axis_index("s") * NC + lax.axis_index("c")
        b = lax.shift_right_logical(wid, 2)
        jq = wid & 3
        j0 = jq * JW

        pltpu.sync_copy(xt_hbm.at[pl.ds((b * M + j0) * TC * K * 128,
                                        JW * TC * K * 128)], xst)
        pltpu.sync_copy(nlt_hbm.at[pl.ds((b * (M // 8) + jq * 2) * TC * 8 * 128,
                                         2 * TC * 8 * 128)], nlst)
        for hi in range(TC):
            pltpu.sync_copy(it_hbm.at[pl.ds((hi * B + b) * 128, 128)],
                            itv.at[pl.ds(hi * 128, 128)])
        pltpu.sync_copy(n_hbm, nv)

        iota = lax.iota(jnp.int32, L)

        def tj_body(t, _):
            jr = lax.shift_right_logical(t, 5)
            hi = lax.shift_right_logical(t, 3) & 3
            g = t & 7
            jgr = lax.shift_right_logical(jr, 3)
            js = jr & 7
            nlc = nlst[pl.ds(((jgr * TC + hi) * 8 + js) * 128 + g * L, L)]
            tv = plsc.load_gather(itv, [jnp.maximum(nlc - 1, 0)])
            tv = jnp.where(nlc == 0, 0, tv)
            tjb[pl.ds((jr * TC + hi) * 128 + g * L, L)] = tv * D
            return 0

        lax.fori_loop(0, JW * TC * 8, tj_body, 0, unroll=False)

        def ii_body(t, _):
            hi = lax.shift_right_logical(t, 3)
            g = t & 7
            til = itv[pl.ds(hi * 128 + g * L, L)]
            ibase = til * D
            for p in range(D):
                iitab[pl.ds((p * TC + hi) * 128 + g * L, L)] = (
                    plsc.load_gather(nv, [ibase + p]))
            return 0

        lax.fori_loop(0, TC * 8, ii_body, 0, unroll=False)

        def fill(buf, jr, pq0, npq):
            def f_body(t, _):
                hi = lax.shift_right_logical(t, 3)
                g = t & 7
                dyn = hi * 128 + g * L
                tjv = tjb[pl.ds(jr * TC * 128 + dyn, L)]
                jj = [plsc.load_gather(nv, [tjv + q]) for q in range(D)]
                xk = [xst[pl.ds((jr * TC + hi) * LANE + k * 128 + g * L, L)]
                      for k in range(K)]
                need_p = sorted({(pq0 + r) // D for r in range(npq)})
                ii = {p: iitab[pl.ds(p * TC * 128 + dyn, L)] for p in need_p}
                kdyn = hi * LANE + g * L
                for r in range(npq):
                    pq = pq0 + r
                    tpq = ii[pq // D] * jj[pq % D]
                    for k in range(K):
                        buf[pl.ds(r * TC * LANE + k * 128 + kdyn, L)] = tpq * xk[k]
                return 0

            lax.fori_loop(0, TC * 8, f_body, 0, unroll=False)

        def j_body(jr, _):
            obase = ((b * M + j0 + jr) * DD) * TC * K * 128

            @pl.when(jr > 0)
            def _():
                pltpu.make_async_copy(
                    bufa, out_hbm.at[pl.ds(0, PQA * TC * K * 128)], sema).wait()

            fill(bufa, jr, 0, PQA)
            pltpu.async_copy(
                bufa, out_hbm.at[pl.ds(obase, PQA * TC * K * 128)], sema)

            @pl.when(jr > 0)
            def _():
                pltpu.make_async_copy(
                    bufb, out_hbm.at[pl.ds(0, PQB * TC * K * 128)], semb).wait()

            fill(bufb, jr, PQA, PQB)
            pltpu.async_copy(
                bufb, out_hbm.at[pl.ds(obase + PQA * TC * K * 128,
                                       PQB * TC * K * 128)], semb)
            return 0

        lax.fori_loop(0, JW, j_body, 0, unroll=False)

        pltpu.make_async_copy(
            bufa, out_hbm.at[pl.ds(0, PQA * TC * K * 128)], sema).wait()
        pltpu.make_async_copy(
            bufb, out_hbm.at[pl.ds(0, PQB * TC * K * 128)], semb).wait()

    out_flat = sc_kernel(xt, nlt, itt, n_pad)
    out6 = out_flat.reshape(B, M, DD, TC, K, 128)
    return out6.transpose(0, 3, 5, 1, 4, 2).reshape(B, N, M, K, DD)


# speedup vs baseline: 1.0140x; 1.0140x over previous
import functools

import jax
import jax.numpy as jnp
from jax import lax
from jax.experimental import pallas as pl
from jax.experimental.pallas import tpu as pltpu
from jax.experimental.pallas import tpu_sc as plsc

NC, NS, L = 2, 16, 16
NW = NC * NS


def kernel(x, Imagetype, neighbor_list, n):
    B, N, M, K = x.shape
    NT, D = n.shape
    DD = D * D
    TC = N // 128
    JW = M // 4
    PQA = 13
    PQB = DD - PQA
    LANE = K * 128
    ROWJ = DD * TC * LANE

    xt = x.reshape(B, TC, 128, M, K).transpose(0, 3, 1, 4, 2).reshape(-1)
    nlt = neighbor_list.reshape(B, TC, 128, M // 8, 8).transpose(0, 3, 1, 4, 2).reshape(-1)
    it_flat = Imagetype.reshape(B * N)
    n_pad = jnp.pad(n.reshape(NT * D), (0, 256 - NT * D))

    mesh = plsc.VectorSubcoreMesh(core_axis_name="c", subcore_axis_name="s")

    @functools.partial(
        pl.kernel,
        out_type=jax.ShapeDtypeStruct((B * M * DD * TC * K * 128,), jnp.float32),
        mesh=mesh,
        compiler_params=pltpu.CompilerParams(needs_layout_passes=False),
        scratch_types=[
            pltpu.VMEM((JW * TC * K * 128,), jnp.float32),
            pltpu.VMEM((2 * TC * 8 * 128,), jnp.int32),
            pltpu.VMEM((N,), jnp.int32),
            pltpu.VMEM((256,), jnp.float32),
            pltpu.VMEM((JW * TC * 128,), jnp.int32),
            pltpu.VMEM((D * TC * 128,), jnp.float32),
            pltpu.VMEM((PQA * TC * K * 128,), jnp.float32),
            pltpu.VMEM((PQB * TC * K * 128,), jnp.float32),
            pltpu.SemaphoreType.DMA,
            pltpu.SemaphoreType.DMA,
        ],
    )
    def sc_kernel(xt_hbm, nlt_hbm, it_hbm, n_hbm, out_hbm,
                  xst, nlst, itv, nv, tjb, iitab, bufa, bufb, sema, semb):
        wid = lax.axis_index("s") * NC + lax.axis_index("c")
        b = lax.shift_right_logical(wid, 2)
        jq = wid & 3
        j0 = jq * JW

        pltpu.sync_copy(xt_hbm.at[pl.ds((b * M + j0) * TC * K * 128,
                                        JW * TC * K * 128)], xst)
        pltpu.sync_copy(nlt_hbm.at[pl.ds((b * (M // 8) + jq * 2) * TC * 8 * 128,
                                         2 * TC * 8 * 128)], nlst)
        pltpu.sync_copy(it_hbm.at[pl.ds(b * N, N)], itv)
        pltpu.sync_copy(n_hbm, nv)

        iota = lax.iota(jnp.int32, L)

        def tj_body(t, _):
            jr = lax.shift_right_logical(t, 5)
            hi = lax.shift_right_logical(t, 3) & 3
            g = t & 7
            jgr = lax.shift_right_logical(jr, 3)
            js = jr & 7
            nlc = nlst[pl.ds(((jgr * TC + hi) * 8 + js) * 128 + g * L, L)]
            tv = plsc.load_gather(itv, [jnp.maximum(nlc - 1, 0)])
            tv = jnp.where(nlc == 0, 0, tv)
            tjb[pl.ds((jr * TC + hi) * 128 + g * L, L)] = tv * D
            return 0

        lax.fori_loop(0, JW * TC * 8, tj_body, 0, unroll=False)

        def ii_body(t, _):
            hi = lax.shift_right_logical(t, 3)
            g = t & 7
            til = itv[pl.ds(hi * 128 + g * L, L)]
            ibase = til * D
            for p in range(D):
                iitab[pl.ds((p * TC + hi) * 128 + g * L, L)] = (
                    plsc.load_gather(nv, [ibase + p]))
            return 0

        lax.fori_loop(0, TC * 8, ii_body, 0, unroll=False)

        def fill(buf, jr, pq0, npq):
            def f_body(t, _):
                hi = lax.shift_right_logical(t, 3)
                g = t & 7
                dyn = hi * 128 + g * L
                tjv = tjb[pl.ds(jr * TC * 128 + dyn, L)]
                jj = [plsc.load_gather(nv, [tjv + q]) for q in range(D)]
                xk = [xst[pl.ds((jr * TC + hi) * LANE + k * 128 + g * L, L)]
                      for k in range(K)]
                need_p = sorted({(pq0 + r) // D for r in range(npq)})
                ii = {p: iitab[pl.ds(p * TC * 128 + dyn, L)] for p in need_p}
                kdyn = hi * LANE + g * L
                for r in range(npq):
                    pq = pq0 + r
                    tpq = ii[pq // D] * jj[pq % D]
                    for k in range(K):
                        buf[pl.ds(r * TC * LANE + k * 128 + kdyn, L)] = tpq * xk[k]
                return 0

            lax.fori_loop(0, TC * 8, f_body, 0, unroll=False)

        def j_body(jr, _):
            obase = ((b * M + j0 + jr) * DD) * TC * K * 128

            @pl.when(jr > 0)
            def _():
                pltpu.make_async_copy(
                    bufa, out_hbm.at[pl.ds(0, PQA * TC * K * 128)], sema).wait()

            fill(bufa, jr, 0, PQA)
            pltpu.async_copy(
                bufa, out_hbm.at[pl.ds(obase, PQA * TC * K * 128)], sema)

            @pl.when(jr > 0)
            def _():
                pltpu.make_async_copy(
                    bufb, out_hbm.at[pl.ds(0, PQB * TC * K * 128)], semb).wait()

            fill(bufb, jr, PQA, PQB)
            pltpu.async_copy(
                bufb, out_hbm.at[pl.ds(obase + PQA * TC * K * 128,
                                       PQB * TC * K * 128)], semb)
            return 0

        lax.fori_loop(0, JW, j_body, 0, unroll=False)

        pltpu.make_async_copy(
            bufa, out_hbm.at[pl.ds(0, PQA * TC * K * 128)], sema).wait()
        pltpu.make_async_copy(
            bufb, out_hbm.at[pl.ds(0, PQB * TC * K * 128)], semb).wait()

    out_flat = sc_kernel(xt, nlt, it_flat, n_pad)
    out6 = out_flat.reshape(B, M, DD, TC, K, 128)
    return out6.transpose(0, 3, 5, 1, 4, 2).reshape(B, N, M, K, DD)


# speedup vs baseline: 1.0149x; 1.0009x over previous
import functools

import jax
import jax.numpy as jnp
from jax import lax
from jax.experimental import pallas as pl
from jax.experimental.pallas import tpu as pltpu
from jax.experimental.pallas import tpu_sc as plsc

NC, NS, L = 2, 16, 16
NW = NC * NS


def kernel(x, Imagetype, neighbor_list, n):
    B, N, M, K = x.shape
    NT, D = n.shape
    DD = D * D
    TC = N // 128
    JW = M // 4
    PQA = 13
    PQB = DD - PQA
    LANE = K * 128
    ROWJ = DD * TC * LANE

    xt = x.reshape(B, TC, 128, M, K).transpose(0, 3, 1, 4, 2).reshape(-1)
    nlt = neighbor_list.reshape(B, TC, 128, M // 8, 8).transpose(0, 3, 1, 4, 2).reshape(-1)
    it_flat = Imagetype.reshape(B * N)
    n_pad = jnp.pad(n.reshape(NT * D), (0, 256 - NT * D))

    mesh = plsc.VectorSubcoreMesh(core_axis_name="c", subcore_axis_name="s")

    @functools.partial(
        pl.kernel,
        out_type=jax.ShapeDtypeStruct((B * M * DD * TC * K * 128,), jnp.float32),
        mesh=mesh,
        compiler_params=pltpu.CompilerParams(needs_layout_passes=False),
        scratch_types=[
            pltpu.VMEM((JW * TC * K * 128,), jnp.float32),
            pltpu.VMEM((2 * TC * 8 * 128,), jnp.int32),
            pltpu.VMEM((N,), jnp.int32),
            pltpu.VMEM((256,), jnp.float32),
            pltpu.VMEM((JW * TC * 128,), jnp.int32),
            pltpu.VMEM((D * TC * 128,), jnp.float32),
            pltpu.VMEM((PQA * TC * K * 128,), jnp.float32),
            pltpu.VMEM((PQB * TC * K * 128,), jnp.float32),
            pltpu.SemaphoreType.DMA,
            pltpu.SemaphoreType.DMA,
        ],
    )
    def sc_kernel(xt_hbm, nlt_hbm, it_hbm, n_hbm, out_hbm,
                  xst, nlst, itv, nv, tjb, iitab, bufa, bufb, sema, semb):
        wid = lax.axis_index("s") * NC + lax.axis_index("c")
        b = lax.shift_right_logical(wid, 2)
        jq = wid & 3
        j0 = jq * JW

        pltpu.sync_copy(xt_hbm.at[pl.ds((b * M + j0) * TC * K * 128,
                                        JW * TC * K * 128)], xst)
        pltpu.sync_copy(nlt_hbm.at[pl.ds((b * (M // 8) + jq * 2) * TC * 8 * 128,
                                         2 * TC * 8 * 128)], nlst)
        pltpu.sync_copy(it_hbm.at[pl.ds(b * N, N)], itv)
        pltpu.sync_copy(n_hbm, nv)

        def tj_body(t, _):
            jr = lax.shift_right_logical(t, 5)
            hi = lax.shift_right_logical(t, 3) & 3
            g = t & 7
            jgr = lax.shift_right_logical(jr, 3)
            js = jr & 7
            nlc = nlst[pl.ds(((jgr * TC + hi) * 8 + js) * 128 + g * L, L)]
            tv = plsc.load_gather(itv, [jnp.maximum(nlc - 1, 0)])
            tv = jnp.where(nlc == 0, 0, tv)
            tjb[pl.ds((jr * TC + hi) * 128 + g * L, L)] = tv * D
            return 0

        lax.fori_loop(0, JW * TC * 8, tj_body, 0, unroll=False)

        def ii_body(t, _):
            hi = lax.shift_right_logical(t, 3)
            g = t & 7
            til = itv[pl.ds(hi * 128 + g * L, L)]
            ibase = til * D
            for p in range(D):
                iitab[pl.ds((p * TC + hi) * 128 + g * L, L)] = (
                    plsc.load_gather(nv, [ibase + p]))
            return 0

        lax.fori_loop(0, TC * 8, ii_body, 0, unroll=False)

        def fill(buf, jr, pq0, npq):
            def f_body(t, _):
                hi = lax.shift_right_logical(t, 3)
                g = t & 7
                dyn = hi * 128 + g * L
                tjv = tjb[pl.ds(jr * TC * 128 + dyn, L)]
                jj = [plsc.load_gather(nv, [tjv + q]) for q in range(D)]
                xk = [xst[pl.ds((jr * TC + hi) * LANE + k * 128 + g * L, L)]
                      for k in range(K)]
                need_p = sorted({(pq0 + r) // D for r in range(npq)})
                ii = {p: iitab[pl.ds(p * TC * 128 + dyn, L)] for p in need_p}
                kdyn = hi * LANE + g * L
                for r in range(npq):
                    pq = pq0 + r
                    tpq = ii[pq // D] * jj[pq % D]
                    for k in range(K):
                        buf[pl.ds(r * TC * LANE + k * 128 + kdyn, L)] = tpq * xk[k]
                return 0

            lax.fori_loop(0, TC * 8, f_body, 0, unroll=False)

        def j_body(jr, _):
            obase = ((b * M + j0 + jr) * DD) * TC * K * 128

            @pl.when(jr > 0)
            def _():
                pltpu.make_async_copy(
                    bufa, out_hbm.at[pl.ds(0, PQA * TC * K * 128)], sema).wait()

            fill(bufa, jr, 0, PQA)
            pltpu.async_copy(
                bufa, out_hbm.at[pl.ds(obase, PQA * TC * K * 128)], sema)

            @pl.when(jr > 0)
            def _():
                pltpu.make_async_copy(
                    bufb, out_hbm.at[pl.ds(0, PQB * TC * K * 128)], semb).wait()

            fill(bufb, jr, PQA, PQB)
            pltpu.async_copy(
                bufb, out_hbm.at[pl.ds(obase + PQA * TC * K * 128,
                                       PQB * TC * K * 128)], semb)
            return 0

        lax.fori_loop(0, JW, j_body, 0, unroll=False)

        pltpu.make_async_copy(
            bufa, out_hbm.at[pl.ds(0, PQA * TC * K * 128)], sema).wait()
        pltpu.make_async_copy(
            bufb, out_hbm.at[pl.ds(0, PQB * TC * K * 128)], semb).wait()

    out_flat = sc_kernel(xt, nlt, it_flat, n_pad)
    out6 = out_flat.reshape(B, M, DD, TC, K, 128)
    return out6.transpose(0, 3, 5, 1, 4, 2).reshape(B, N, M, K, DD)


# speedup vs baseline: 1.0306x; 1.0155x over previous
import functools

import jax
import jax.numpy as jnp
from jax import lax
from jax.experimental import pallas as pl
from jax.experimental.pallas import tpu as pltpu
from jax.experimental.pallas import tpu_sc as plsc

NC, NS, L = 2, 16, 16
NW = NC * NS


def kernel(x, Imagetype, neighbor_list, n):
    B, N, M, K = x.shape
    NT, D = n.shape
    DD = D * D
    TC = N // 128
    JW = M // 4
    PQA = 13
    PQB = DD - PQA
    LANE = K * 128
    ROWJ = DD * TC * LANE

    xt = x.reshape(B, TC, 128, M, K).transpose(0, 3, 1, 4, 2).reshape(-1)
    nlt = neighbor_list.reshape(B, TC, 128, M // 8, 8).transpose(0, 3, 1, 4, 2).reshape(-1)
    it_flat = Imagetype.reshape(B * N)
    n_pad = jnp.pad(n.reshape(NT * D), (0, 256 - NT * D))

    mesh = plsc.VectorSubcoreMesh(core_axis_name="c", subcore_axis_name="s")

    @functools.partial(
        pl.kernel,
        out_type=jax.ShapeDtypeStruct((B * M * DD * TC * K * 128,), jnp.float32),
        mesh=mesh,
        compiler_params=pltpu.CompilerParams(needs_layout_passes=False),
        scratch_types=[
            pltpu.VMEM((JW * TC * K * 128,), jnp.float32),
            pltpu.VMEM((2 * TC * 8 * 128,), jnp.int32),
            pltpu.VMEM((N,), jnp.int32),
            pltpu.VMEM((256,), jnp.float32),
            pltpu.VMEM((JW * TC * 128,), jnp.int32),
            pltpu.VMEM((D * TC * 128,), jnp.float32),
            pltpu.VMEM((PQA * TC * K * 128,), jnp.float32),
            pltpu.VMEM((PQB * TC * K * 128,), jnp.float32),
            pltpu.SemaphoreType.DMA,
            pltpu.SemaphoreType.DMA,
        ],
    )
    def sc_kernel(xt_hbm, nlt_hbm, it_hbm, n_hbm, out_hbm,
                  xst, nlst, itv, nv, tjb, iitab, bufa, bufb, sema, semb):
        wid = lax.axis_index("s") * NC + lax.axis_index("c")
        b = lax.shift_right_logical(wid, 2)
        jq = wid & 3
        j0 = jq * JW

        xcp = pltpu.async_copy(
            xt_hbm.at[pl.ds((b * M + j0) * TC * K * 128, JW * TC * K * 128)],
            xst, sema)
        pltpu.sync_copy(nlt_hbm.at[pl.ds((b * (M // 8) + jq * 2) * TC * 8 * 128,
                                         2 * TC * 8 * 128)], nlst)
        pltpu.sync_copy(it_hbm.at[pl.ds(b * N, N)], itv)
        pltpu.sync_copy(n_hbm, nv)

        def tj_body(t, _):
            jr = lax.shift_right_logical(t, 5)
            hi = lax.shift_right_logical(t, 3) & 3
            g = t & 7
            jgr = lax.shift_right_logical(jr, 3)
            js = jr & 7
            nlc = nlst[pl.ds(((jgr * TC + hi) * 8 + js) * 128 + g * L, L)]
            tv = plsc.load_gather(itv, [jnp.maximum(nlc - 1, 0)])
            tv = jnp.where(nlc == 0, 0, tv)
            tjb[pl.ds((jr * TC + hi) * 128 + g * L, L)] = tv * D
            return 0

        lax.fori_loop(0, JW * TC * 8, tj_body, 0, unroll=False)

        def ii_body(t, _):
            hi = lax.shift_right_logical(t, 3)
            g = t & 7
            til = itv[pl.ds(hi * 128 + g * L, L)]
            ibase = til * D
            for p in range(D):
                iitab[pl.ds((p * TC + hi) * 128 + g * L, L)] = (
                    plsc.load_gather(nv, [ibase + p]))
            return 0

        lax.fori_loop(0, TC * 8, ii_body, 0, unroll=False)
        xcp.wait()

        def fill(buf, jr, pq0, npq):
            def f_body(t, _):
                hi = lax.shift_right_logical(t, 3)
                g = t & 7
                dyn = hi * 128 + g * L
                tjv = tjb[pl.ds(jr * TC * 128 + dyn, L)]
                jj = [plsc.load_gather(nv, [tjv + q]) for q in range(D)]
                xk = [xst[pl.ds((jr * TC + hi) * LANE + k * 128 + g * L, L)]
                      for k in range(K)]
                need_p = sorted({(pq0 + r) // D for r in range(npq)})
                ii = {p: iitab[pl.ds(p * TC * 128 + dyn, L)] for p in need_p}
                kdyn = hi * LANE + g * L
                for r in range(npq):
                    pq = pq0 + r
                    tpq = ii[pq // D] * jj[pq % D]
                    for k in range(K):
                        buf[pl.ds(r * TC * LANE + k * 128 + kdyn, L)] = tpq * xk[k]
                return 0

            lax.fori_loop(0, TC * 8, f_body, 0, unroll=False)

        def j_body(jr, _):
            obase = ((b * M + j0 + jr) * DD) * TC * K * 128

            @pl.when(jr > 0)
            def _():
                pltpu.make_async_copy(
                    bufa, out_hbm.at[pl.ds(0, PQA * TC * K * 128)], sema).wait()

            fill(bufa, jr, 0, PQA)
            pltpu.async_copy(
                bufa, out_hbm.at[pl.ds(obase, PQA * TC * K * 128)], sema)

            @pl.when(jr > 0)
            def _():
                pltpu.make_async_copy(
                    bufb, out_hbm.at[pl.ds(0, PQB * TC * K * 128)], semb).wait()

            fill(bufb, jr, PQA, PQB)
            pltpu.async_copy(
                bufb, out_hbm.at[pl.ds(obase + PQA * TC * K * 128,
                                       PQB * TC * K * 128)], semb)
            return 0

        lax.fori_loop(0, JW, j_body, 0, unroll=False)

        pltpu.make_async_copy(
            bufa, out_hbm.at[pl.ds(0, PQA * TC * K * 128)], sema).wait()
        pltpu.make_async_copy(
            bufb, out_hbm.at[pl.ds(0, PQB * TC * K * 128)], semb).wait()

    out_flat = sc_kernel(xt, nlt, it_flat, n_pad)
    out6 = out_flat.reshape(B, M, DD, TC, K, 128)
    return out6.transpose(0, 3, 5, 1, 4, 2).reshape(B, N, M, K, DD)
